# baseline (speedup 1.0000x reference)
import functools

import jax
import jax.numpy as jnp
from jax import lax
from jax.experimental import pallas as pl
from jax.experimental.pallas import tpu as pltpu
from jax.experimental.pallas import tpu_sc as plsc

_LANES = 16


def _make_sc_kernel(B, T, D, NW, NC):
    bw = B // NW
    Tp = (T + 7) // 8 * 8
    Th = Tp // 2
    mesh = plsc.VectorSubcoreMesh(core_axis_name="c", subcore_axis_name="s")

    @functools.partial(
        pl.kernel,
        mesh=mesh,
        compiler_params=pltpu.CompilerParams(disable_bounds_checks=True),
        out_type=jax.ShapeDtypeStruct((B, T, D), jnp.float32),
        scratch_types=[
            pltpu.VMEM((bw, Tp), jnp.int32),
            pltpu.VMEM((Tp, D), jnp.float32),
            pltpu.VMEM((Tp, D), jnp.float32),
            pltpu.SemaphoreType.DMA,
            pltpu.SemaphoreType.DMA,
            pltpu.SemaphoreType.DMA,
            pltpu.SemaphoreType.DMA,
        ],
    )
    def sc_kernel(tok_hbm, table_hbm, pos_hbm, out_hbm,
                  tokw_v, buf0, buf1, gsem0, gsem1, ssem0, ssem1):
        c = lax.axis_index("c")
        s = lax.axis_index("s")
        w = s * NC + c
        base = w * bw
        bufs = (buf0, buf1)
        gsems = (gsem0, gsem1)
        ssems = (ssem0, ssem1)
        pltpu.sync_copy(tok_hbm.at[pl.ds(base, bw), :], tokw_v)

        def start_gather(j, b):
            pltpu.async_copy(table_hbm.at[tokw_v.at[j]], bufs[b], gsems[b])

        def wait_gather(b):
            pltpu.make_async_copy(table_hbm.at[pl.ds(0, Tp), :], bufs[b],
                                  gsems[b]).wait()

        def start_scatter(j, b):
            for h in range(2):
                off = pl.multiple_of(h * Th + w * 0, 8)
                pltpu.async_copy(bufs[b].at[pl.ds(h * Th, Th), :],
                                 out_hbm.at[base + j, pl.ds(off, Th), :],
                                 ssems[b])

        def wait_scatter(b):
            pltpu.make_async_copy(table_hbm.at[pl.ds(0, Tp), :], bufs[b],
                                  ssems[b]).wait()

        start_gather(0, 0)

        @pl.loop(0, bw, step=2)
        def pair(j0):
            @pl.when(j0 > 0)
            def _():
                wait_scatter(1)
            start_gather(j0 + 1, 1)
            wait_gather(0)
            start_scatter(j0, 0)
            wait_scatter(0)
            @pl.when(j0 + 2 < bw)
            def _():
                start_gather(j0 + 2, 0)
            wait_gather(1)
            start_scatter(j0 + 1, 1)

        wait_scatter(1)

    return sc_kernel


def kernel(tokens, token_table, position_embedding):
    B, T = tokens.shape
    V, D = token_table.shape
    NW = 32
    NC = 2
    assert B % NW == 0 and D % _LANES == 0
    Tp = (T + 7) // 8 * 8
    tok = jnp.pad(tokens.astype(jnp.int32), ((0, 0), (0, Tp - T)))
    sc = _make_sc_kernel(B, T, D, NW, NC)
    return sc(tok, token_table, position_embedding)
